# Initial kernel scaffold; baseline (speedup 1.0000x reference)
import jax
import jax.numpy as jnp
from jax.experimental import pallas as pl


def kernel(coords, row_emb, col_emb):
    raise NotImplementedError("write your pallas kernel here")


# speedup vs baseline: 7.8608x; 7.8608x over previous
import functools

import jax
import jax.numpy as jnp
from jax import lax
from jax.experimental import pallas as pl
from jax.experimental.pallas import tpu as pltpu
from jax.experimental.pallas import tpu_sc as plsc

_S = 19
_D = 128
_DH = 64
_NC, _NS, _L = 2, 16, 16
_NW = _NC * _NS
_B, _LN = 4096, 200
_N = _B * _LN
_PW = _N // _NW
_CH = 128
_NCH = _PW // _CH


def _table_body(row_ref, col_ref, out_ref):
  r = jnp.broadcast_to(row_ref[...][:, None, :], (_S, _S, _DH))
  c = jnp.broadcast_to(col_ref[...][None, :, :], (_S, _S, _DH))
  out_ref[...] = jnp.concatenate([r, c], axis=-1)


def _build_table(row_emb, col_emb):
  return pl.pallas_call(
      _table_body,
      out_shape=jax.ShapeDtypeStruct((_S, _S, _D), jnp.float32),
  )(row_emb, col_emb).reshape(_S * _S, _D)


def _sc_body(r_hbm, c_hbm, tab_hbm, out_hbm, r_v, c_v, idx_v, rows_v, gsem):
  wid = lax.axis_index("s") * _NC + lax.axis_index("c")
  base0 = wid * _PW
  pltpu.sync_copy(r_hbm.at[pl.ds(base0, _PW)], r_v)
  pltpu.sync_copy(c_hbm.at[pl.ds(base0, _PW)], c_v)

  def idx_outer(k, _):
    def idx_inner(j, _):
      pos = (k * (_CH // _L) + j) * _L
      rv = r_v[pl.ds(pos, _L)]
      cv = c_v[pl.ds(pos, _L)]
      iv = jnp.maximum(rv, 0) * _S + jnp.maximum(cv, 0)
      idx_v[k, pl.ds(j * _L, _L)] = iv
      return 0
    lax.fori_loop(0, _CH // _L, idx_inner, 0)
    return 0
  lax.fori_loop(0, _NCH, idx_outer, 0)

  def chunk(k, _):
    pltpu.async_copy(tab_hbm.at[idx_v.at[k]], rows_v, gsem).wait()
    pltpu.sync_copy(rows_v, out_hbm.at[pl.ds(base0 + k * _CH, _CH)])
    return 0
  lax.fori_loop(0, _NCH, chunk, 0)


_sc_gather = functools.partial(
    pl.kernel,
    out_type=jax.ShapeDtypeStruct((_N, _D), jnp.float32),
    mesh=plsc.VectorSubcoreMesh(
        core_axis_name="c", subcore_axis_name="s",
        num_cores=_NC, num_subcores=_NS),
    scratch_types=[
        pltpu.VMEM((_PW,), jnp.int32),
        pltpu.VMEM((_PW,), jnp.int32),
        pltpu.VMEM((_NCH, _CH), jnp.int32),
        pltpu.VMEM((_CH, _D), jnp.float32),
        pltpu.SemaphoreType.DMA,
    ],
)(_sc_body)


def kernel(coords, row_emb, col_emb):
  table = _build_table(row_emb, col_emb)
  r = coords[..., 0].reshape(_N)
  c = coords[..., 1].reshape(_N)
  out = _sc_gather(r, c, table)
  return out.reshape(_B, _LN, _D)


# speedup vs baseline: 7.9066x; 1.0058x over previous
import functools

import jax
import jax.numpy as jnp
from jax import lax
from jax.experimental import pallas as pl
from jax.experimental.pallas import tpu as pltpu
from jax.experimental.pallas import tpu_sc as plsc

_S = 19
_D = 128
_DH = 64
_NC, _NS, _L = 2, 16, 16
_NW = _NC * _NS
_B, _LN = 4096, 200
_N = _B * _LN
_PW = _N // _NW
_CH = 128
_NCH = _PW // _CH


def _table_body(row_ref, col_ref, out_ref):
  r = jnp.broadcast_to(row_ref[...][:, None, :], (_S, _S, _DH))
  c = jnp.broadcast_to(col_ref[...][None, :, :], (_S, _S, _DH))
  out_ref[...] = jnp.concatenate([r, c], axis=-1)


def _build_table(row_emb, col_emb):
  return pl.pallas_call(
      _table_body,
      out_shape=jax.ShapeDtypeStruct((_S, _S, _D), jnp.float32),
  )(row_emb, col_emb).reshape(_S * _S, _D)


_NBUF = 4
_SEG = 3200
_NSEG = _PW // _SEG
_CPS = _SEG // _CH


def _sc_body(r_hbm, c_hbm, tab_hbm, out_hbm,
             r_v, c_v, idx_v, rows_v, gsem, ssem):
  wid = lax.axis_index("s") * _NC + lax.axis_index("c")
  base0 = wid * _PW

  def seg(s, _):
    pltpu.sync_copy(r_hbm.at[pl.ds(base0 + s * _SEG, _SEG)], r_v)
    pltpu.sync_copy(c_hbm.at[pl.ds(base0 + s * _SEG, _SEG)], c_v)
    def per_chunk(t, _):
      def per_vec(j, _):
        pos = t * _CH + j * _L
        iv = (jnp.maximum(r_v[pl.ds(pos, _L)], 0) * _S
              + jnp.maximum(c_v[pl.ds(pos, _L)], 0))
        idx_v[s * _CPS + t, pl.ds(j * _L, _L)] = iv
        return 0
      lax.fori_loop(0, _CH // _L, per_vec, 0)
      return 0
    lax.fori_loop(0, _CPS, per_chunk, 0)
    return 0
  lax.fori_loop(0, _NSEG, seg, 0)

  def out_at(k):
    return out_hbm.at[pl.ds(base0 + k * _CH, _CH)]

  def rnd(i, _):
    descs = []
    for b in range(_NBUF):
      k = i * _NBUF + b
      def wait_prev(b=b, k=k):
        pltpu.make_async_copy(rows_v.at[b], out_at(k - _NBUF), ssem.at[b]).wait()
      pl.when(i > 0)(wait_prev)
      descs.append(
          pltpu.async_copy(tab_hbm.at[idx_v.at[k]], rows_v.at[b], gsem.at[b]))
    for b in range(_NBUF):
      k = i * _NBUF + b
      descs[b].wait()
      pltpu.async_copy(rows_v.at[b], out_at(k), ssem.at[b])
    return 0
  lax.fori_loop(0, _NCH // _NBUF, rnd, 0)
  for b in range(_NBUF):
    k = _NCH - _NBUF + b
    pltpu.make_async_copy(rows_v.at[b], out_at(k), ssem.at[b]).wait()


_sc_gather = functools.partial(
    pl.kernel,
    out_type=jax.ShapeDtypeStruct((_N, _D), jnp.float32),
    mesh=plsc.VectorSubcoreMesh(
        core_axis_name="c", subcore_axis_name="s",
        num_cores=_NC, num_subcores=_NS),
    scratch_types=[
        pltpu.VMEM((_SEG,), jnp.int32),
        pltpu.VMEM((_SEG,), jnp.int32),
        pltpu.VMEM((_NCH, _CH), jnp.int32),
        pltpu.VMEM((_NBUF, _CH, _D), jnp.float32),
        pltpu.SemaphoreType.DMA((_NBUF,)),
        pltpu.SemaphoreType.DMA((_NBUF,)),
    ],
)(_sc_body)


def kernel(coords, row_emb, col_emb):
  table = _build_table(row_emb, col_emb)
  r = coords[..., 0].reshape(_N)
  c = coords[..., 1].reshape(_N)
  out = _sc_gather(r, c, table)
  return out.reshape(_B, _LN, _D)


# speedup vs baseline: 18.7832x; 2.3756x over previous
import functools

import jax
import jax.numpy as jnp
from jax import lax
from jax.experimental import pallas as pl
from jax.experimental.pallas import tpu as pltpu
from jax.experimental.pallas import tpu_sc as plsc

_S = 19
_D = 128
_DH = 64
_NC, _NS, _L = 2, 16, 16
_NW = _NC * _NS
_B, _LN = 4096, 200
_N = _B * _LN
_PW = _N // _NW
_CH = 128
_NCH = _PW // _CH


def _table_body(row_ref, col_ref, out_ref):
  r = jnp.broadcast_to(row_ref[...][:, None, :], (_S, _S, _DH))
  c = jnp.broadcast_to(col_ref[...][None, :, :], (_S, _S, _DH))
  out_ref[...] = jnp.concatenate([r, c], axis=-1)


def _build_table(row_emb, col_emb):
  return pl.pallas_call(
      _table_body,
      out_shape=jax.ShapeDtypeStruct((_S, _S, _D), jnp.float32),
  )(row_emb, col_emb).reshape(_S * _S, _D)


_NBUF = 2
_SEG = 3200
_NSEG = _PW // _SEG
_CPS = _SEG // _CH


def _sc_body(r_hbm, c_hbm, tab_hbm, out_hbm,
             r_v, c_v, idx_v, tab_v, rows_v, gsem, ssem):
  wid = lax.axis_index("s") * _NC + lax.axis_index("c")
  base0 = wid * _PW

  def load_tab():
    pltpu.sync_copy(tab_hbm, tab_v)
  pl.when(lax.axis_index("s") == 0)(load_tab)
  plsc.subcore_barrier()

  def seg(s, _):
    pltpu.sync_copy(r_hbm.at[pl.ds(base0 + s * _SEG, _SEG)], r_v)
    pltpu.sync_copy(c_hbm.at[pl.ds(base0 + s * _SEG, _SEG)], c_v)
    def per_chunk(t, _):
      def per_vec(j, _):
        pos = t * _CH + j * _L
        iv = (jnp.maximum(r_v[pl.ds(pos, _L)], 0) * _S
              + jnp.maximum(c_v[pl.ds(pos, _L)], 0))
        idx_v[s * _CPS + t, pl.ds(j * _L, _L)] = iv
        return 0
      lax.fori_loop(0, _CH // _L, per_vec, 0)
      return 0
    lax.fori_loop(0, _CPS, per_chunk, 0)
    return 0
  lax.fori_loop(0, _NSEG, seg, 0)

  def out_at(k):
    return out_hbm.at[pl.ds(base0 + k * _CH, _CH)]

  def rnd(i, _):
    descs = []
    for b in range(_NBUF):
      k = i * _NBUF + b
      def wait_prev(b=b, k=k):
        pltpu.make_async_copy(rows_v.at[b], out_at(k - _NBUF), ssem.at[b]).wait()
      pl.when(i > 0)(wait_prev)
      descs.append(
          pltpu.async_copy(tab_v.at[idx_v.at[k]], rows_v.at[b], gsem.at[b]))
    for b in range(_NBUF):
      k = i * _NBUF + b
      descs[b].wait()
      pltpu.async_copy(rows_v.at[b], out_at(k), ssem.at[b])
    return 0
  lax.fori_loop(0, _NCH // _NBUF, rnd, 0)
  for b in range(_NBUF):
    k = _NCH - _NBUF + b
    pltpu.make_async_copy(rows_v.at[b], out_at(k), ssem.at[b]).wait()


_sc_gather = functools.partial(
    pl.kernel,
    out_type=jax.ShapeDtypeStruct((_N, _D), jnp.float32),
    mesh=plsc.VectorSubcoreMesh(
        core_axis_name="c", subcore_axis_name="s",
        num_cores=_NC, num_subcores=_NS),
    scratch_types=[
        pltpu.VMEM((_SEG,), jnp.int32),
        pltpu.VMEM((_SEG,), jnp.int32),
        pltpu.VMEM((_NCH, _CH), jnp.int32),
        pltpu.VMEM_SHARED((_S * _S, _D), jnp.float32),
        pltpu.VMEM((_NBUF, _CH, _D), jnp.float32),
        pltpu.SemaphoreType.DMA((_NBUF,)),
        pltpu.SemaphoreType.DMA((_NBUF,)),
    ],
)(_sc_body)


def kernel(coords, row_emb, col_emb):
  table = _build_table(row_emb, col_emb)
  r = coords[..., 0].reshape(_N)
  c = coords[..., 1].reshape(_N)
  out = _sc_gather(r, c, table)
  return out.reshape(_B, _LN, _D)


# speedup vs baseline: 25.9402x; 1.3810x over previous
import functools

import jax
import jax.numpy as jnp
from jax import lax
from jax.experimental import pallas as pl
from jax.experimental.pallas import tpu as pltpu
from jax.experimental.pallas import tpu_sc as plsc

_S = 19
_D = 128
_DH = 64
_NC, _NS, _L = 2, 16, 16
_NW = _NC * _NS
_B, _LN = 4096, 200
_N = _B * _LN
_PW = _N // _NW
_CH = 128
_NCH = _PW // _CH


def _table_body(row_ref, col_ref, out_ref):
  r = jnp.broadcast_to(row_ref[...][:, None, :], (_S, _S, _DH))
  c = jnp.broadcast_to(col_ref[...][None, :, :], (_S, _S, _DH))
  out_ref[...] = jnp.concatenate([r, c], axis=-1)


def _build_table(row_emb, col_emb):
  return pl.pallas_call(
      _table_body,
      out_shape=jax.ShapeDtypeStruct((_S, _S, _D), jnp.float32),
  )(row_emb, col_emb).reshape(_S * _S, _D)


_NBUF = 4
_SEG = 3200
_NSEG = _PW // _SEG
_CPS = _SEG // _CH


def _sc_body(r_hbm, c_hbm, tab_hbm, out_hbm,
             r_v, c_v, idx_v, tab_v, rows_v, gsem, ssem):
  wid = lax.axis_index("s") * _NC + lax.axis_index("c")
  base0 = wid * _PW

  def load_tab():
    pltpu.sync_copy(tab_hbm, tab_v)
  pl.when(lax.axis_index("s") == 0)(load_tab)
  plsc.subcore_barrier()

  def seg(s, _):
    pltpu.sync_copy(r_hbm.at[pl.ds(base0 + s * _SEG, _SEG)], r_v)
    pltpu.sync_copy(c_hbm.at[pl.ds(base0 + s * _SEG, _SEG)], c_v)
    def per_chunk(t, _):
      def per_vec(j, _):
        pos = t * _CH + j * _L
        iv = (jnp.maximum(r_v[pl.ds(pos, _L)], 0) * _S
              + jnp.maximum(c_v[pl.ds(pos, _L)], 0))
        idx_v[s * _CPS + t, pl.ds(j * _L, _L)] = iv
        return 0
      lax.fori_loop(0, _CH // _L, per_vec, 0)
      return 0
    lax.fori_loop(0, _CPS, per_chunk, 0)
    return 0
  lax.fori_loop(0, _NSEG, seg, 0)

  def out_at(k):
    return out_hbm.at[pl.ds(base0 + k * _CH, _CH)]

  def rnd(i, _):
    descs = []
    for b in range(_NBUF):
      k = i * _NBUF + b
      def wait_prev(b=b, k=k):
        pltpu.make_async_copy(rows_v.at[b], out_at(k - _NBUF), ssem.at[b]).wait()
      pl.when(i > 0)(wait_prev)
      descs.append(
          pltpu.async_copy(tab_v.at[idx_v.at[k]], rows_v.at[b], gsem.at[b]))
    for b in range(_NBUF):
      k = i * _NBUF + b
      descs[b].wait()
      pltpu.async_copy(rows_v.at[b], out_at(k), ssem.at[b])
    return 0
  lax.fori_loop(0, _NCH // _NBUF, rnd, 0)
  for b in range(_NBUF):
    k = _NCH - _NBUF + b
    pltpu.make_async_copy(rows_v.at[b], out_at(k), ssem.at[b]).wait()


_sc_gather = functools.partial(
    pl.kernel,
    out_type=jax.ShapeDtypeStruct((_N, _D), jnp.float32),
    mesh=plsc.VectorSubcoreMesh(
        core_axis_name="c", subcore_axis_name="s",
        num_cores=_NC, num_subcores=_NS),
    scratch_types=[
        pltpu.VMEM((_SEG,), jnp.int32),
        pltpu.VMEM((_SEG,), jnp.int32),
        pltpu.VMEM((_NCH, _CH), jnp.int32),
        pltpu.VMEM_SHARED((_S * _S, _D), jnp.float32),
        pltpu.VMEM((_NBUF, _CH, _D), jnp.float32),
        pltpu.SemaphoreType.DMA((_NBUF,)),
        pltpu.SemaphoreType.DMA((_NBUF,)),
    ],
)(_sc_body)


def kernel(coords, row_emb, col_emb):
  table = _build_table(row_emb, col_emb)
  r = coords[..., 0].reshape(_N)
  c = coords[..., 1].reshape(_N)
  out = _sc_gather(r, c, table)
  return out.reshape(_B, _LN, _D)


# speedup vs baseline: 26.1565x; 1.0083x over previous
import functools

import jax
import jax.numpy as jnp
from jax import lax
from jax.experimental import pallas as pl
from jax.experimental.pallas import tpu as pltpu
from jax.experimental.pallas import tpu_sc as plsc

_S = 19
_D = 128
_DH = 64
_NC, _NS, _L = 2, 16, 16
_NW = _NC * _NS
_B, _LN = 4096, 200
_N = _B * _LN
_PW = _N // _NW
_CH = 64
_NCH = _PW // _CH


def _table_body(row_ref, col_ref, out_ref):
  r = jnp.broadcast_to(row_ref[...][:, None, :], (_S, _S, _DH))
  c = jnp.broadcast_to(col_ref[...][None, :, :], (_S, _S, _DH))
  out_ref[...] = jnp.concatenate([r, c], axis=-1)


def _build_table(row_emb, col_emb):
  return pl.pallas_call(
      _table_body,
      out_shape=jax.ShapeDtypeStruct((_S, _S, _D), jnp.float32),
  )(row_emb, col_emb).reshape(_S * _S, _D)


_NBUF = 8
_SEG = 3200
_NSEG = _PW // _SEG
_CPS = _SEG // _CH


def _sc_body(r_hbm, c_hbm, tab_hbm, out_hbm,
             r_v, c_v, idx_v, tab_v, rows_v, gsem, ssem):
  wid = lax.axis_index("s") * _NC + lax.axis_index("c")
  base0 = wid * _PW

  def load_tab():
    pltpu.sync_copy(tab_hbm, tab_v)
  pl.when(lax.axis_index("s") == 0)(load_tab)
  plsc.subcore_barrier()

  def seg(s, _):
    pltpu.sync_copy(r_hbm.at[pl.ds(base0 + s * _SEG, _SEG)], r_v)
    pltpu.sync_copy(c_hbm.at[pl.ds(base0 + s * _SEG, _SEG)], c_v)
    def per_chunk(t, _):
      def per_vec(j, _):
        pos = t * _CH + j * _L
        iv = (jnp.maximum(r_v[pl.ds(pos, _L)], 0) * _S
              + jnp.maximum(c_v[pl.ds(pos, _L)], 0))
        idx_v[s * _CPS + t, pl.ds(j * _L, _L)] = iv
        return 0
      lax.fori_loop(0, _CH // _L, per_vec, 0)
      return 0
    lax.fori_loop(0, _CPS, per_chunk, 0)
    return 0
  lax.fori_loop(0, _NSEG, seg, 0)

  def out_at(k):
    return out_hbm.at[pl.ds(base0 + k * _CH, _CH)]

  def rnd(i, _):
    descs = []
    for b in range(_NBUF):
      k = i * _NBUF + b
      def wait_prev(b=b, k=k):
        pltpu.make_async_copy(rows_v.at[b], out_at(k - _NBUF), ssem.at[b]).wait()
      pl.when(i > 0)(wait_prev)
      descs.append(
          pltpu.async_copy(tab_v.at[idx_v.at[k]], rows_v.at[b], gsem.at[b]))
    for b in range(_NBUF):
      k = i * _NBUF + b
      descs[b].wait()
      pltpu.async_copy(rows_v.at[b], out_at(k), ssem.at[b])
    return 0
  lax.fori_loop(0, _NCH // _NBUF, rnd, 0)
  for b in range(_NBUF):
    k = _NCH - _NBUF + b
    pltpu.make_async_copy(rows_v.at[b], out_at(k), ssem.at[b]).wait()


_sc_gather = functools.partial(
    pl.kernel,
    out_type=jax.ShapeDtypeStruct((_N, _D), jnp.float32),
    mesh=plsc.VectorSubcoreMesh(
        core_axis_name="c", subcore_axis_name="s",
        num_cores=_NC, num_subcores=_NS),
    scratch_types=[
        pltpu.VMEM((_SEG,), jnp.int32),
        pltpu.VMEM((_SEG,), jnp.int32),
        pltpu.VMEM((_NCH, _CH), jnp.int32),
        pltpu.VMEM_SHARED((_S * _S, _D), jnp.float32),
        pltpu.VMEM((_NBUF, _CH, _D), jnp.float32),
        pltpu.SemaphoreType.DMA((_NBUF,)),
        pltpu.SemaphoreType.DMA((_NBUF,)),
    ],
)(_sc_body)


def kernel(coords, row_emb, col_emb):
  table = _build_table(row_emb, col_emb)
  r = coords[..., 0].reshape(_N)
  c = coords[..., 1].reshape(_N)
  out = _sc_gather(r, c, table)
  return out.reshape(_B, _LN, _D)
